# baseline (speedup 1.0000x reference)
import functools

import jax
import jax.numpy as jnp
from jax import lax
from jax.experimental import pallas as pl
from jax.experimental.pallas import tpu as pltpu
from jax.experimental.pallas import tpu_sc as plsc

N = 10000
C = 128
T = 12
E = 320000
NC = 2
NS = 16
NW = NC * NS
N_ACC = 10112
CK = 64
CPW = 160
E_PAD = NW * CPW * CK
NB = 2
RPW = 624
ZPW = N_ACC // NS
BR = 2000
NR = N // BR

_mesh = plsc.VectorSubcoreMesh(core_axis_name="c", subcore_axis_name="s")


def _copy_out(acc, out, s, base):
    pltpu.sync_copy(acc.at[pl.ds(s * RPW, RPW)], out.at[pl.ds(base + s * RPW, RPW)])

    @pl.when(s == 0)
    def _():
        pltpu.sync_copy(acc.at[pl.ds(NS * RPW, N - NS * RPW)],
                        out.at[pl.ds(base + NS * RPW, N - NS * RPW)])


@functools.partial(
    pl.kernel,
    out_type=jax.ShapeDtypeStruct((NC * N, C), jnp.float32),
    mesh=_mesh,
    scratch_types=[
        pltpu.VMEM_SHARED((N_ACC, C), jnp.float32),
        pltpu.VMEM((CPW // 2, CK), jnp.int32),
        pltpu.VMEM((CPW // 2, CK), jnp.int32),
        pltpu.VMEM((NB, CK, C), jnp.float32),
        pltpu.SemaphoreType.DMA,
        pltpu.SemaphoreType.DMA,
        pltpu.SemaphoreType.DMA,
        pltpu.SemaphoreType.DMA,
    ],
)
def _prop_call(table, src32, dst32, zeros, out, acc, idxs, idxd, rows,
               g0, g1, p0, p1):
    c = lax.axis_index("c")
    s = lax.axis_index("s")
    wid = c * NS + s
    gsems = [g0, g1]
    ssems = [p0, p1]
    hc = CPW // 2
    pltpu.sync_copy(zeros.at[pl.ds(s * ZPW, ZPW)], acc.at[pl.ds(s * ZPW, ZPW)])
    plsc.subcore_barrier()
    for half in range(2):
        pltpu.sync_copy(src32.at[wid, pl.ds(half * hc, hc)], idxs)
        pltpu.sync_copy(dst32.at[wid, pl.ds(half * hc, hc)], idxd)
        gd = [pltpu.async_copy(table.at[idxs.at[b]], rows.at[b], gsems[b])
              for b in range(NB)]

        @pl.loop(0, hc // NB - 1)
        def _edges(i):
            j0 = i * NB
            sd = []
            for b in range(NB):
                gd[b].wait()
                sd.append(pltpu.async_copy(rows.at[b], acc.at[idxd.at[j0 + b]],
                                           ssems[b], add=True))
            for b in range(NB):
                sd[b].wait()
                pltpu.async_copy(table.at[idxs.at[j0 + NB + b]], rows.at[b],
                                 gsems[b])

        for b in range(NB):
            gd[b].wait()
            pltpu.sync_copy(rows.at[b], acc.at[idxd.at[hc - NB + b]], add=True)

    plsc.subcore_barrier()
    _copy_out(acc, out, s, c * N)


@functools.partial(
    pl.kernel,
    out_type=jax.ShapeDtypeStruct((T * NC * N, C), jnp.float32),
    mesh=_mesh,
    scratch_types=[
        pltpu.VMEM_SHARED((N_ACC, C), jnp.float32),
        pltpu.VMEM((CPW // 2, CK), jnp.int32),
        pltpu.VMEM((CPW // 2, CK), jnp.int32),
        pltpu.VMEM((NB, CK, C), jnp.float32),
        pltpu.SemaphoreType.DMA,
        pltpu.SemaphoreType.DMA,
        pltpu.SemaphoreType.DMA,
        pltpu.SemaphoreType.DMA,
    ],
)
def _xall_call(xs_flat, src12, dst32, zeros, out, acc,
               idxs, idxd, rows, g0, g1, p0, p1):
    c = lax.axis_index("c")
    s = lax.axis_index("s")
    wid = c * NS + s
    gsems = [g0, g1]
    ssems = [p0, p1]
    hc = CPW // 2
    for p in range(T):
        table = xs_flat
        pltpu.sync_copy(zeros.at[pl.ds(s * ZPW, ZPW)], acc.at[pl.ds(s * ZPW, ZPW)])
        plsc.subcore_barrier()
        for half in range(2):
            pltpu.sync_copy(src12.at[p, wid, pl.ds(half * hc, hc)], idxs)
            pltpu.sync_copy(dst32.at[wid, pl.ds(half * hc, hc)], idxd)
            gd = [pltpu.async_copy(table.at[idxs.at[b]], rows.at[b], gsems[b])
                  for b in range(NB)]

            @pl.loop(0, hc // NB - 1)
            def _edges(i):
                j0 = i * NB
                sd = []
                for b in range(NB):
                    gd[b].wait()
                    sd.append(pltpu.async_copy(rows.at[b], acc.at[idxd.at[j0 + b]],
                                               ssems[b], add=True))
                for b in range(NB):
                    sd[b].wait()
                    pltpu.async_copy(table.at[idxs.at[j0 + NB + b]], rows.at[b],
                                     gsems[b])

            for b in range(NB):
                gd[b].wait()
                pltpu.sync_copy(rows.at[b], acc.at[idxd.at[hc - NB + b]], add=True)

        plsc.subcore_barrier()
        _copy_out(acc, out, s, (p * NC + c) * N)
        if p < T - 1:
            plsc.subcore_barrier()


def _dinv_block(hist_ref):
    deg = hist_ref[0, :, 0:1] + hist_ref[1, :, 0:1]
    return jnp.where(deg > 0.0, lax.rsqrt(deg), 0.0)


def _scale_body(x_ref, hist_ref, o_ref):
    dinv = _dinv_block(hist_ref)
    o_ref[...] = x_ref[...] * dinv[None]


_scale_call = pl.pallas_call(
    _scale_body,
    grid=(T, NR),
    in_specs=[
        pl.BlockSpec((1, BR, C), lambda t, r: (t, r, 0)),
        pl.BlockSpec((NC, BR, C), lambda t, r: (0, r, 0)),
    ],
    out_specs=pl.BlockSpec((1, BR, C), lambda t, r: (t, r, 0)),
    out_shape=jax.ShapeDtypeStruct((T, N, C), jnp.float32),
)


def _mm(a, b):
    return jnp.dot(a, b, preferred_element_type=jnp.float32)


def _step0_body(x_ref, gx_ref, hist_ref, w0_ref, w1_ref, b_ref, fcw_ref, fcb_ref,
                hn_ref, hs_ref, y_ref):
    dinv = _dinv_block(hist_ref)
    x = x_ref[...]
    gxs = -dinv * (gx_ref[0] + gx_ref[1])
    u = _mm(x, w0_ref[...]) + _mm(gxs, w1_ref[...]) + b_ref[...]
    z = jax.nn.sigmoid(u[:, :C])
    ht = jnp.tanh(u[:, C:])
    hn = (1.0 - z) * ht
    hn_ref[...] = hn
    hs_ref[...] = dinv * hn
    y = jnp.sum(hn * fcw_ref[...], axis=1) + fcb_ref[0, 0]
    y_ref[...] = jnp.broadcast_to(y[None, None], (1, 8, BR))


_step0_call = pl.pallas_call(
    _step0_body,
    grid=(NR,),
    in_specs=[
        pl.BlockSpec((BR, C), lambda r: (r, 0)),
        pl.BlockSpec((NC, BR, C), lambda r: (0, r, 0)),
        pl.BlockSpec((NC, BR, C), lambda r: (0, r, 0)),
        pl.BlockSpec((C, 2 * C), lambda r: (0, 0)),
        pl.BlockSpec((C, 2 * C), lambda r: (0, 0)),
        pl.BlockSpec((1, 2 * C), lambda r: (0, 0)),
        pl.BlockSpec((1, C), lambda r: (0, 0)),
        pl.BlockSpec((1, 1), lambda r: (0, 0)),
    ],
    out_specs=[
        pl.BlockSpec((BR, C), lambda r: (r, 0)),
        pl.BlockSpec((BR, C), lambda r: (r, 0)),
        pl.BlockSpec((1, 8, BR), lambda r: (r, 0, 0)),
    ],
    out_shape=[
        jax.ShapeDtypeStruct((N, C), jnp.float32),
        jax.ShapeDtypeStruct((N, C), jnp.float32),
        jax.ShapeDtypeStruct((NR, 8, BR), jnp.float32),
    ],
)


def _gates_body(x_ref, gx_ref, hist_ref, h_ref, gh_ref,
                w0x_ref, w1x_ref, w0h_ref, w1h_ref, b_ref, z_ref, hr_ref, hrs_ref):
    dinv = _dinv_block(hist_ref)
    x = x_ref[0]
    h = h_ref[...]
    gxs = -dinv * (gx_ref[0] + gx_ref[1])
    ghs = -dinv * (gh_ref[0] + gh_ref[1])
    u = (_mm(x, w0x_ref[...]) + _mm(gxs, w1x_ref[...])
         + _mm(h, w0h_ref[...]) + _mm(ghs, w1h_ref[...]) + b_ref[...])
    z = jax.nn.sigmoid(u[:, :C])
    r = jax.nn.sigmoid(u[:, C:])
    hr = h * r
    z_ref[...] = z
    hr_ref[...] = hr
    hrs_ref[...] = dinv * hr


def _make_gates(t):
    return pl.pallas_call(
        _gates_body,
        grid=(NR,),
        in_specs=[
            pl.BlockSpec((1, BR, C), lambda r: (0, r, 0)),
            pl.BlockSpec((NC, BR, C), lambda r, _t=t: (_t, r, 0)),
        pl.BlockSpec((NC, BR, C), lambda r: (0, r, 0)),
        pl.BlockSpec((BR, C), lambda r: (r, 0)),
            pl.BlockSpec((NC, BR, C), lambda r: (0, r, 0)),
            pl.BlockSpec((C, 2 * C), lambda r: (0, 0)),
            pl.BlockSpec((C, 2 * C), lambda r: (0, 0)),
            pl.BlockSpec((C, 2 * C), lambda r: (0, 0)),
            pl.BlockSpec((C, 2 * C), lambda r: (0, 0)),
            pl.BlockSpec((1, 2 * C), lambda r: (0, 0)),
        ],
        out_specs=[
            pl.BlockSpec((BR, C), lambda r: (r, 0)),
            pl.BlockSpec((BR, C), lambda r: (r, 0)),
            pl.BlockSpec((BR, C), lambda r: (r, 0)),
        ],
        out_shape=[
            jax.ShapeDtypeStruct((N, C), jnp.float32),
            jax.ShapeDtypeStruct((N, C), jnp.float32),
            jax.ShapeDtypeStruct((N, C), jnp.float32),
        ],
    )


def _update_body(x_ref, gx_ref, hist_ref, h_ref, z_ref, hr_ref, gr_ref,
                 w0x_ref, w1x_ref, w0h_ref, w1h_ref, b_ref, fcw_ref, fcb_ref,
                 hn_ref, hs_ref, y_ref):
    dinv = _dinv_block(hist_ref)
    x = x_ref[0]
    h = h_ref[...]
    z = z_ref[...]
    gxs = -dinv * (gx_ref[0] + gx_ref[1])
    grs = -dinv * (gr_ref[0] + gr_ref[1])
    ht = jnp.tanh(_mm(x, w0x_ref[...]) + _mm(gxs, w1x_ref[...])
                  + _mm(hr_ref[...], w0h_ref[...]) + _mm(grs, w1h_ref[...]) + b_ref[...])
    hn = z * h + (1.0 - z) * ht
    hn_ref[...] = hn
    hs_ref[...] = dinv * hn
    y = jnp.sum(hn * fcw_ref[...], axis=1) + fcb_ref[0, 0]
    y_ref[...] = jnp.broadcast_to(y[None, None], (1, 8, BR))


def _make_update(t):
    return pl.pallas_call(
        _update_body,
        grid=(NR,),
        in_specs=[
            pl.BlockSpec((1, BR, C), lambda r: (0, r, 0)),
            pl.BlockSpec((NC, BR, C), lambda r, _t=t: (_t, r, 0)),
            pl.BlockSpec((NC, BR, C), lambda r: (0, r, 0)),
            pl.BlockSpec((BR, C), lambda r: (r, 0)),
            pl.BlockSpec((BR, C), lambda r: (r, 0)),
            pl.BlockSpec((BR, C), lambda r: (r, 0)),
            pl.BlockSpec((NC, BR, C), lambda r: (0, r, 0)),
            pl.BlockSpec((C, C), lambda r: (0, 0)),
            pl.BlockSpec((C, C), lambda r: (0, 0)),
            pl.BlockSpec((C, C), lambda r: (0, 0)),
            pl.BlockSpec((C, C), lambda r: (0, 0)),
            pl.BlockSpec((1, C), lambda r: (0, 0)),
            pl.BlockSpec((1, C), lambda r: (0, 0)),
            pl.BlockSpec((1, 1), lambda r: (0, 0)),
        ],
        out_specs=[
            pl.BlockSpec((BR, C), lambda r: (r, 0)),
            pl.BlockSpec((BR, C), lambda r: (r, 0)),
            pl.BlockSpec((1, 8, BR), lambda r: (r, 0, 0)),
        ],
        out_shape=[
            jax.ShapeDtypeStruct((N, C), jnp.float32),
            jax.ShapeDtypeStruct((N, C), jnp.float32),
            jax.ShapeDtypeStruct((NR, 8, BR), jnp.float32),
        ],
    )


def kernel(X, edge_index, params):
    X3 = X.reshape(T, N, C)
    src = edge_index[0]
    dst = edge_index[1]

    npad = E_PAD - E
    ar = jnp.arange(npad, dtype=jnp.int32)
    src_p = jnp.concatenate([src, (ar * 1301) % N])
    dst_p = jnp.concatenate([dst, N + (ar % (N_ACC - N))])
    srch_p = jnp.concatenate([src, N + (ar % (N_ACC - N))])
    src32 = src_p.reshape(NW, CPW, CK)
    dst32 = dst_p.reshape(NW, CPW, CK)
    src32h = srch_p.reshape(NW, CPW, CK)
    zeros_c = jnp.zeros((N_ACC, C), jnp.float32)
    ones_c = jnp.ones((N_ACC, C), jnp.float32)

    p = params
    w0_zr = jnp.concatenate([p["Whz"][0], p["Whr"][0]], axis=1)
    w1_zr = jnp.concatenate([p["Whz"][1], p["Whr"][1]], axis=1)
    w0x_zr = jnp.concatenate([p["Wxz"][0], p["Wxr"][0]], axis=1)
    w1x_zr = jnp.concatenate([p["Wxz"][1], p["Wxr"][1]], axis=1)
    b_zr = jnp.concatenate([p["bxz"] + p["bhz"], p["bxr"] + p["bhr"]])[None]
    w0_0 = jnp.concatenate([p["Wxz"][0], p["Wxh"][0]], axis=1)
    w1_0 = jnp.concatenate([p["Wxz"][1], p["Wxh"][1]], axis=1)
    b_0 = jnp.concatenate([p["bxz"] + p["bhz"], p["bxh"] + p["bhh"]])[None]
    b_h = (p["bxh"] + p["bhh"])[None]
    fcw = p["fc_W"].reshape(1, C)
    fcb = p["fc_b"].reshape(1, 1)

    src12 = src32[None] + (jnp.arange(T, dtype=jnp.int32) * N)[:, None, None, None]

    hist = _prop_call(ones_c, src32h, src32h, zeros_c).reshape(NC, N, C)
    Xs = _scale_call(X3, hist)
    Gall = _xall_call(Xs.reshape(T * N, C), src12, dst32, zeros_c)
    Gall = Gall.reshape(T * NC, N, C)

    ys = []
    H, Hs, y0 = _step0_call(X3[0], Gall, hist, w0_0, w1_0, b_0, fcw, fcb)
    ys.append(y0[:, 0, :].reshape(N))
    for t in range(1, T):
        Gh = _prop_call(Hs, src32, dst32, zeros_c).reshape(NC, N, C)
        Z, HR, HRs = _make_gates(t)(X3[t:t + 1], Gall, hist, H, Gh,
                                    w0x_zr, w1x_zr, w0_zr, w1_zr, b_zr)
        Gr = _prop_call(HRs, src32, dst32, zeros_c).reshape(NC, N, C)
        H, Hs, yt = _make_update(t)(X3[t:t + 1], Gall, hist, H, Z, HR, Gr,
                                    p["Wxh"][0], p["Wxh"][1], p["Whh"][0], p["Whh"][1],
                                    b_h, fcw, fcb)
        ys.append(yt[:, 0, :].reshape(N))
    Y = jnp.stack(ys)
    return Y.reshape(1, T, N, 1)


# speedup vs baseline: 1.2573x; 1.2573x over previous
import functools

import jax
import jax.numpy as jnp
from jax import lax
from jax.experimental import pallas as pl
from jax.experimental.pallas import tpu as pltpu
from jax.experimental.pallas import tpu_sc as plsc

N = 10000
C = 128
T = 12
E = 320000
NC = 2
NS = 16
NW = NC * NS
N_ACC = 10112
CK = 40
CPW = 256
E_PAD = NW * CPW * CK
NB = 4
NF = 4
RPW = 624
ZPW = N_ACC // NS
BR = 2000
NR = N // BR

_mesh = plsc.VectorSubcoreMesh(core_axis_name="c", subcore_axis_name="s")


def _copy_out(acc, out, s, base):
    pltpu.sync_copy(acc.at[pl.ds(s * RPW, RPW)], out.at[pl.ds(base + s * RPW, RPW)])

    @pl.when(s == 0)
    def _():
        pltpu.sync_copy(acc.at[pl.ds(NS * RPW, N - NS * RPW)],
                        out.at[pl.ds(base + NS * RPW, N - NS * RPW)])


@functools.partial(
    pl.kernel,
    out_type=jax.ShapeDtypeStruct((NC * N, C), jnp.float32),
    mesh=_mesh,
    scratch_types=[
        pltpu.VMEM_SHARED((N_ACC, C), jnp.float32),
        pltpu.VMEM((CPW // NF, CK), jnp.int32),
        pltpu.VMEM((CPW // NF, CK), jnp.int32),
        pltpu.VMEM((NB, CK, C), jnp.float32),
    ] + [pltpu.SemaphoreType.DMA] * (2 * NB),
)
def _prop_call(table, src32, dst32, zeros, out, acc, idxs, idxd, rows, *sems):
    c = lax.axis_index("c")
    s = lax.axis_index("s")
    wid = c * NS + s
    gsems = sems[:NB]
    ssems = sems[NB:]
    hc = CPW // NF
    pltpu.sync_copy(zeros.at[pl.ds(s * ZPW, ZPW)], acc.at[pl.ds(s * ZPW, ZPW)])
    plsc.subcore_barrier()
    for half in range(NF):
        pltpu.sync_copy(src32.at[wid, pl.ds(half * hc, hc)], idxs)
        pltpu.sync_copy(dst32.at[wid, pl.ds(half * hc, hc)], idxd)
        gd = [pltpu.async_copy(table.at[idxs.at[b]], rows.at[b], gsems[b])
              for b in range(NB)]

        @pl.loop(0, hc // NB - 1)
        def _edges(i):
            j0 = i * NB
            sd = []
            for b in range(NB):
                gd[b].wait()
                sd.append(pltpu.async_copy(rows.at[b], acc.at[idxd.at[j0 + b]],
                                           ssems[b], add=True))
            for b in range(NB):
                sd[b].wait()
                pltpu.async_copy(table.at[idxs.at[j0 + NB + b]], rows.at[b],
                                 gsems[b])

        for b in range(NB):
            gd[b].wait()
            pltpu.sync_copy(rows.at[b], acc.at[idxd.at[hc - NB + b]], add=True)

    plsc.subcore_barrier()
    _copy_out(acc, out, s, c * N)


@functools.partial(
    pl.kernel,
    out_type=jax.ShapeDtypeStruct((T * NC * N, C), jnp.float32),
    mesh=_mesh,
    scratch_types=[
        pltpu.VMEM_SHARED((N_ACC, C), jnp.float32),
        pltpu.VMEM((CPW // NF, CK), jnp.int32),
        pltpu.VMEM((CPW // NF, CK), jnp.int32),
        pltpu.VMEM((NB, CK, C), jnp.float32),
    ] + [pltpu.SemaphoreType.DMA] * (2 * NB),
)
def _xall_call(xs_flat, src12, dst32, zeros, out, acc, idxs, idxd, rows, *sems):
    c = lax.axis_index("c")
    s = lax.axis_index("s")
    wid = c * NS + s
    gsems = sems[:NB]
    ssems = sems[NB:]
    hc = CPW // NF
    for p in range(T):
        table = xs_flat
        pltpu.sync_copy(zeros.at[pl.ds(s * ZPW, ZPW)], acc.at[pl.ds(s * ZPW, ZPW)])
        plsc.subcore_barrier()
        for half in range(NF):
            pltpu.sync_copy(src12.at[p, wid, pl.ds(half * hc, hc)], idxs)
            pltpu.sync_copy(dst32.at[wid, pl.ds(half * hc, hc)], idxd)
            gd = [pltpu.async_copy(table.at[idxs.at[b]], rows.at[b], gsems[b])
                  for b in range(NB)]

            @pl.loop(0, hc // NB - 1)
            def _edges(i):
                j0 = i * NB
                sd = []
                for b in range(NB):
                    gd[b].wait()
                    sd.append(pltpu.async_copy(rows.at[b], acc.at[idxd.at[j0 + b]],
                                               ssems[b], add=True))
                for b in range(NB):
                    sd[b].wait()
                    pltpu.async_copy(table.at[idxs.at[j0 + NB + b]], rows.at[b],
                                     gsems[b])

            for b in range(NB):
                gd[b].wait()
                pltpu.sync_copy(rows.at[b], acc.at[idxd.at[hc - NB + b]], add=True)

        plsc.subcore_barrier()
        _copy_out(acc, out, s, (p * NC + c) * N)
        if p < T - 1:
            plsc.subcore_barrier()


def _dinv_block(hist_ref):
    deg = hist_ref[0, :, 0:1] + hist_ref[1, :, 0:1]
    return jnp.where(deg > 0.0, lax.rsqrt(deg), 0.0)


def _scale_body(x_ref, hist_ref, o_ref):
    dinv = _dinv_block(hist_ref)
    o_ref[...] = x_ref[...] * dinv[None]


_scale_call = pl.pallas_call(
    _scale_body,
    grid=(T, NR),
    in_specs=[
        pl.BlockSpec((1, BR, C), lambda t, r: (t, r, 0)),
        pl.BlockSpec((NC, BR, C), lambda t, r: (0, r, 0)),
    ],
    out_specs=pl.BlockSpec((1, BR, C), lambda t, r: (t, r, 0)),
    out_shape=jax.ShapeDtypeStruct((T, N, C), jnp.float32),
)


def _mm(a, b):
    return jnp.dot(a, b, preferred_element_type=jnp.float32)


def _step0_body(x_ref, gx_ref, hist_ref, w0_ref, w1_ref, b_ref, fcw_ref, fcb_ref,
                hn_ref, hs_ref, y_ref):
    dinv = _dinv_block(hist_ref)
    x = x_ref[...]
    gxs = -dinv * (gx_ref[0] + gx_ref[1])
    u = _mm(x, w0_ref[...]) + _mm(gxs, w1_ref[...]) + b_ref[...]
    z = jax.nn.sigmoid(u[:, :C])
    ht = jnp.tanh(u[:, C:])
    hn = (1.0 - z) * ht
    hn_ref[...] = hn
    hs_ref[...] = dinv * hn
    y = jnp.sum(hn * fcw_ref[...], axis=1) + fcb_ref[0, 0]
    y_ref[...] = jnp.broadcast_to(y[None, None], (1, 8, BR))


_step0_call = pl.pallas_call(
    _step0_body,
    grid=(NR,),
    in_specs=[
        pl.BlockSpec((BR, C), lambda r: (r, 0)),
        pl.BlockSpec((NC, BR, C), lambda r: (0, r, 0)),
        pl.BlockSpec((NC, BR, C), lambda r: (0, r, 0)),
        pl.BlockSpec((C, 2 * C), lambda r: (0, 0)),
        pl.BlockSpec((C, 2 * C), lambda r: (0, 0)),
        pl.BlockSpec((1, 2 * C), lambda r: (0, 0)),
        pl.BlockSpec((1, C), lambda r: (0, 0)),
        pl.BlockSpec((1, 1), lambda r: (0, 0)),
    ],
    out_specs=[
        pl.BlockSpec((BR, C), lambda r: (r, 0)),
        pl.BlockSpec((BR, C), lambda r: (r, 0)),
        pl.BlockSpec((1, 8, BR), lambda r: (r, 0, 0)),
    ],
    out_shape=[
        jax.ShapeDtypeStruct((N, C), jnp.float32),
        jax.ShapeDtypeStruct((N, C), jnp.float32),
        jax.ShapeDtypeStruct((NR, 8, BR), jnp.float32),
    ],
)


def _gates_body(x_ref, gx_ref, hist_ref, h_ref, gh_ref,
                w0x_ref, w1x_ref, w0h_ref, w1h_ref, b_ref, z_ref, hr_ref, hrs_ref):
    dinv = _dinv_block(hist_ref)
    x = x_ref[0]
    h = h_ref[...]
    gxs = -dinv * (gx_ref[0] + gx_ref[1])
    ghs = -dinv * (gh_ref[0] + gh_ref[1])
    u = (_mm(x, w0x_ref[...]) + _mm(gxs, w1x_ref[...])
         + _mm(h, w0h_ref[...]) + _mm(ghs, w1h_ref[...]) + b_ref[...])
    z = jax.nn.sigmoid(u[:, :C])
    r = jax.nn.sigmoid(u[:, C:])
    hr = h * r
    z_ref[...] = z
    hr_ref[...] = hr
    hrs_ref[...] = dinv * hr


def _make_gates(t):
    return pl.pallas_call(
        _gates_body,
        grid=(NR,),
        in_specs=[
            pl.BlockSpec((1, BR, C), lambda r: (0, r, 0)),
            pl.BlockSpec((NC, BR, C), lambda r, _t=t: (_t, r, 0)),
        pl.BlockSpec((NC, BR, C), lambda r: (0, r, 0)),
        pl.BlockSpec((BR, C), lambda r: (r, 0)),
            pl.BlockSpec((NC, BR, C), lambda r: (0, r, 0)),
            pl.BlockSpec((C, 2 * C), lambda r: (0, 0)),
            pl.BlockSpec((C, 2 * C), lambda r: (0, 0)),
            pl.BlockSpec((C, 2 * C), lambda r: (0, 0)),
            pl.BlockSpec((C, 2 * C), lambda r: (0, 0)),
            pl.BlockSpec((1, 2 * C), lambda r: (0, 0)),
        ],
        out_specs=[
            pl.BlockSpec((BR, C), lambda r: (r, 0)),
            pl.BlockSpec((BR, C), lambda r: (r, 0)),
            pl.BlockSpec((BR, C), lambda r: (r, 0)),
        ],
        out_shape=[
            jax.ShapeDtypeStruct((N, C), jnp.float32),
            jax.ShapeDtypeStruct((N, C), jnp.float32),
            jax.ShapeDtypeStruct((N, C), jnp.float32),
        ],
    )


def _update_body(x_ref, gx_ref, hist_ref, h_ref, z_ref, hr_ref, gr_ref,
                 w0x_ref, w1x_ref, w0h_ref, w1h_ref, b_ref, fcw_ref, fcb_ref,
                 hn_ref, hs_ref, y_ref):
    dinv = _dinv_block(hist_ref)
    x = x_ref[0]
    h = h_ref[...]
    z = z_ref[...]
    gxs = -dinv * (gx_ref[0] + gx_ref[1])
    grs = -dinv * (gr_ref[0] + gr_ref[1])
    ht = jnp.tanh(_mm(x, w0x_ref[...]) + _mm(gxs, w1x_ref[...])
                  + _mm(hr_ref[...], w0h_ref[...]) + _mm(grs, w1h_ref[...]) + b_ref[...])
    hn = z * h + (1.0 - z) * ht
    hn_ref[...] = hn
    hs_ref[...] = dinv * hn
    y = jnp.sum(hn * fcw_ref[...], axis=1) + fcb_ref[0, 0]
    y_ref[...] = jnp.broadcast_to(y[None, None], (1, 8, BR))


def _make_update(t):
    return pl.pallas_call(
        _update_body,
        grid=(NR,),
        in_specs=[
            pl.BlockSpec((1, BR, C), lambda r: (0, r, 0)),
            pl.BlockSpec((NC, BR, C), lambda r, _t=t: (_t, r, 0)),
            pl.BlockSpec((NC, BR, C), lambda r: (0, r, 0)),
            pl.BlockSpec((BR, C), lambda r: (r, 0)),
            pl.BlockSpec((BR, C), lambda r: (r, 0)),
            pl.BlockSpec((BR, C), lambda r: (r, 0)),
            pl.BlockSpec((NC, BR, C), lambda r: (0, r, 0)),
            pl.BlockSpec((C, C), lambda r: (0, 0)),
            pl.BlockSpec((C, C), lambda r: (0, 0)),
            pl.BlockSpec((C, C), lambda r: (0, 0)),
            pl.BlockSpec((C, C), lambda r: (0, 0)),
            pl.BlockSpec((1, C), lambda r: (0, 0)),
            pl.BlockSpec((1, C), lambda r: (0, 0)),
            pl.BlockSpec((1, 1), lambda r: (0, 0)),
        ],
        out_specs=[
            pl.BlockSpec((BR, C), lambda r: (r, 0)),
            pl.BlockSpec((BR, C), lambda r: (r, 0)),
            pl.BlockSpec((1, 8, BR), lambda r: (r, 0, 0)),
        ],
        out_shape=[
            jax.ShapeDtypeStruct((N, C), jnp.float32),
            jax.ShapeDtypeStruct((N, C), jnp.float32),
            jax.ShapeDtypeStruct((NR, 8, BR), jnp.float32),
        ],
    )


def kernel(X, edge_index, params):
    X3 = X.reshape(T, N, C)
    src = edge_index[0]
    dst = edge_index[1]

    npad = E_PAD - E
    ar = jnp.arange(npad, dtype=jnp.int32)
    src_p = jnp.concatenate([src, (ar * 1301) % N])
    dst_p = jnp.concatenate([dst, N + (ar % (N_ACC - N))])
    srch_p = jnp.concatenate([src, N + (ar % (N_ACC - N))])
    src32 = src_p.reshape(NW, CPW, CK)
    dst32 = dst_p.reshape(NW, CPW, CK)
    src32h = srch_p.reshape(NW, CPW, CK)
    zeros_c = jnp.zeros((N_ACC, C), jnp.float32)
    ones_c = jnp.ones((N_ACC, C), jnp.float32)

    p = params
    w0_zr = jnp.concatenate([p["Whz"][0], p["Whr"][0]], axis=1)
    w1_zr = jnp.concatenate([p["Whz"][1], p["Whr"][1]], axis=1)
    w0x_zr = jnp.concatenate([p["Wxz"][0], p["Wxr"][0]], axis=1)
    w1x_zr = jnp.concatenate([p["Wxz"][1], p["Wxr"][1]], axis=1)
    b_zr = jnp.concatenate([p["bxz"] + p["bhz"], p["bxr"] + p["bhr"]])[None]
    w0_0 = jnp.concatenate([p["Wxz"][0], p["Wxh"][0]], axis=1)
    w1_0 = jnp.concatenate([p["Wxz"][1], p["Wxh"][1]], axis=1)
    b_0 = jnp.concatenate([p["bxz"] + p["bhz"], p["bxh"] + p["bhh"]])[None]
    b_h = (p["bxh"] + p["bhh"])[None]
    fcw = p["fc_W"].reshape(1, C)
    fcb = p["fc_b"].reshape(1, 1)

    src12 = src32[None] + (jnp.arange(T, dtype=jnp.int32) * N)[:, None, None, None]

    hist = _prop_call(ones_c, src32h, src32h, zeros_c).reshape(NC, N, C)
    Xs = _scale_call(X3, hist)
    Gall = _xall_call(Xs.reshape(T * N, C), src12, dst32, zeros_c)
    Gall = Gall.reshape(T * NC, N, C)

    ys = []
    H, Hs, y0 = _step0_call(X3[0], Gall, hist, w0_0, w1_0, b_0, fcw, fcb)
    ys.append(y0[:, 0, :].reshape(N))
    for t in range(1, T):
        Gh = _prop_call(Hs, src32, dst32, zeros_c).reshape(NC, N, C)
        Z, HR, HRs = _make_gates(t)(X3[t:t + 1], Gall, hist, H, Gh,
                                    w0x_zr, w1x_zr, w0_zr, w1_zr, b_zr)
        Gr = _prop_call(HRs, src32, dst32, zeros_c).reshape(NC, N, C)
        H, Hs, yt = _make_update(t)(X3[t:t + 1], Gall, hist, H, Z, HR, Gr,
                                    p["Wxh"][0], p["Wxh"][1], p["Whh"][0], p["Whh"][1],
                                    b_h, fcw, fcb)
        ys.append(yt[:, 0, :].reshape(N))
    Y = jnp.stack(ys)
    return Y.reshape(1, T, N, 1)
